# Initial kernel scaffold; baseline (speedup 1.0000x reference)
import jax
import jax.numpy as jnp
from jax.experimental import pallas as pl


def kernel(Attention_map):
    raise NotImplementedError("write your pallas kernel here")


# speedup vs baseline: 18.2186x; 18.2186x over previous
import functools

import jax
import jax.numpy as jnp
import numpy as np
from jax.experimental import pallas as pl
from jax.experimental.pallas import tpu as pltpu

_TOPK_FRAC = 0.2
_ROWS_PER_BLOCK = 8


def _flip_topk_kernel(x_ref, o_ref, *, k, rows, length, idx_bits):
    x = x_ref[...]
    b = jax.lax.bitcast_convert_type(x, jnp.int32)
    s = jnp.where(b < 0, b ^ jnp.int32(0x7FFFFFFF), b)

    kk = jnp.int32(k)

    cnt_nonneg = jnp.sum((s >= 0).astype(jnp.int32), axis=1, keepdims=True)
    base = jnp.where(cnt_nonneg >= kk, jnp.int32(0), jnp.int32(-(2**31)))

    def value_step(i, prefix):
        bit = jnp.int32(1) << (30 - i)
        cand = base | prefix | bit
        cnt = jnp.sum((s >= cand).astype(jnp.int32), axis=1, keepdims=True)
        return jnp.where(cnt >= kk, prefix | bit, prefix)

    prefix = jax.lax.fori_loop(
        0, 31, value_step, jnp.zeros((rows, 1), jnp.int32)
    )
    t = base | prefix

    gt = s > t
    eq = s == t
    cnt_gt = jnp.sum(gt.astype(jnp.int32), axis=1, keepdims=True)
    need = kk - cnt_gt

    ridx = jnp.int32(length - 1) - jax.lax.broadcasted_iota(
        jnp.int32, (rows, length), 1
    )
    keys = jnp.where(eq, ridx, jnp.int32(-1))

    def idx_step(i, pfx):
        bit = jnp.int32(1) << (idx_bits - 1 - i)
        cand = pfx | bit
        cnt = jnp.sum((keys >= cand).astype(jnp.int32), axis=1,
                      keepdims=True)
        return jnp.where(cnt >= need, pfx | bit, pfx)

    pfx = jax.lax.fori_loop(
        0, idx_bits, idx_step, jnp.zeros((rows, 1), jnp.int32)
    )
    flip = gt | (eq & (keys >= pfx))
    o_ref[...] = jnp.where(flip, 1.0 - x, x)


def kernel(Attention_map):
    B, C, H, W = Attention_map.shape
    L = C * H * W
    k = int(np.clip(int(L * _TOPK_FRAC), 1, C))
    idx_bits = max(int(L - 1).bit_length(), 1)

    rows = _ROWS_PER_BLOCK
    flat = Attention_map.reshape(B, L)

    out = pl.pallas_call(
        functools.partial(
            _flip_topk_kernel, k=k, rows=rows, length=L, idx_bits=idx_bits
        ),
        grid=(B // rows,),
        in_specs=[pl.BlockSpec((rows, L), lambda i: (i, 0))],
        out_specs=pl.BlockSpec((rows, L), lambda i: (i, 0)),
        out_shape=jax.ShapeDtypeStruct((B, L), jnp.float32),
        compiler_params=pltpu.CompilerParams(
            dimension_semantics=("arbitrary",),
        ),
    )(flat)
    return out.reshape(B, C, H, W)


# speedup vs baseline: 31.8059x; 1.7458x over previous
import functools

import jax
import jax.numpy as jnp
import numpy as np
from jax.experimental import pallas as pl
from jax.experimental.pallas import tpu as pltpu

_TOPK_FRAC = 0.2
_ROWS_PER_BLOCK = 32


def _flip_topk_kernel(x_ref, o_ref, *, k, rows, length, idx_bits):
    x = x_ref[...]
    b = jax.lax.bitcast_convert_type(x, jnp.int32)
    s = jnp.where(b < 0, b ^ jnp.int32(0x7FFFFFFF), b)

    kk = jnp.int32(k)

    cnt_nonneg = jnp.sum((s >= 0).astype(jnp.int32), axis=1, keepdims=True)
    base = jnp.where(cnt_nonneg >= kk, jnp.int32(0), jnp.int32(-(2**31)))

    def value_step(i, prefix):
        bit = jnp.int32(1) << (30 - i)
        cand = base | prefix | bit
        cnt = jnp.sum((s >= cand).astype(jnp.int32), axis=1, keepdims=True)
        return jnp.where(cnt >= kk, prefix | bit, prefix)

    prefix = jax.lax.fori_loop(
        0, 31, value_step, jnp.zeros((rows, 1), jnp.int32)
    )
    t = base | prefix

    gt = s > t
    eq = s == t
    cnt_gt = jnp.sum(gt.astype(jnp.int32), axis=1, keepdims=True)
    need = kk - cnt_gt

    ridx = jnp.int32(length - 1) - jax.lax.broadcasted_iota(
        jnp.int32, (rows, length), 1
    )
    keys = jnp.where(eq, ridx, jnp.int32(-1))

    def idx_step(i, pfx):
        bit = jnp.int32(1) << (idx_bits - 1 - i)
        cand = pfx | bit
        cnt = jnp.sum((keys >= cand).astype(jnp.int32), axis=1,
                      keepdims=True)
        return jnp.where(cnt >= need, pfx | bit, pfx)

    pfx = jax.lax.fori_loop(
        0, idx_bits, idx_step, jnp.zeros((rows, 1), jnp.int32)
    )
    flip = gt | (eq & (keys >= pfx))
    o_ref[...] = jnp.where(flip, 1.0 - x, x)


def kernel(Attention_map):
    B, C, H, W = Attention_map.shape
    L = C * H * W
    k = int(np.clip(int(L * _TOPK_FRAC), 1, C))
    idx_bits = max(int(L - 1).bit_length(), 1)

    rows = _ROWS_PER_BLOCK
    flat = Attention_map.reshape(B, L)

    out = pl.pallas_call(
        functools.partial(
            _flip_topk_kernel, k=k, rows=rows, length=L, idx_bits=idx_bits
        ),
        grid=(B // rows,),
        in_specs=[pl.BlockSpec((rows, L), lambda i: (i, 0))],
        out_specs=pl.BlockSpec((rows, L), lambda i: (i, 0)),
        out_shape=jax.ShapeDtypeStruct((B, L), jnp.float32),
        compiler_params=pltpu.CompilerParams(
            dimension_semantics=("parallel",),
        ),
    )(flat)
    return out.reshape(B, C, H, W)


# speedup vs baseline: 36.4415x; 1.1457x over previous
import functools

import jax
import jax.numpy as jnp
import numpy as np
from jax.experimental import pallas as pl
from jax.experimental.pallas import tpu as pltpu

_TOPK_FRAC = 0.2
_ROWS_PER_BLOCK = 64


def _flip_topk_kernel(x_ref, o_ref, *, k, rows, length, idx_bits):
    x = x_ref[...]
    b = jax.lax.bitcast_convert_type(x, jnp.int32)
    s = jnp.where(b < 0, b ^ jnp.int32(0x7FFFFFFF), b)

    kk = jnp.int32(k)

    cnt_nonneg = jnp.sum((s >= 0).astype(jnp.int32), axis=1, keepdims=True)
    base = jnp.where(cnt_nonneg >= kk, jnp.int32(0), jnp.int32(-(2**31)))

    def value_step(i, prefix):
        bit = jnp.int32(1) << (30 - i)
        cand = base | prefix | bit
        cnt = jnp.sum((s >= cand).astype(jnp.int32), axis=1, keepdims=True)
        return jnp.where(cnt >= kk, prefix | bit, prefix)

    prefix = jax.lax.fori_loop(
        0, 31, value_step, jnp.zeros((rows, 1), jnp.int32)
    )
    t = base | prefix

    gt = s > t
    eq = s == t
    cnt_gt = jnp.sum(gt.astype(jnp.int32), axis=1, keepdims=True)
    need = kk - cnt_gt

    ridx = jnp.int32(length - 1) - jax.lax.broadcasted_iota(
        jnp.int32, (rows, length), 1
    )
    keys = jnp.where(eq, ridx, jnp.int32(-1))

    def idx_step(i, pfx):
        bit = jnp.int32(1) << (idx_bits - 1 - i)
        cand = pfx | bit
        cnt = jnp.sum((keys >= cand).astype(jnp.int32), axis=1,
                      keepdims=True)
        return jnp.where(cnt >= need, pfx | bit, pfx)

    pfx = jax.lax.fori_loop(
        0, idx_bits, idx_step, jnp.zeros((rows, 1), jnp.int32)
    )
    flip = gt | (eq & (keys >= pfx))
    o_ref[...] = jnp.where(flip, 1.0 - x, x)


def kernel(Attention_map):
    B, C, H, W = Attention_map.shape
    L = C * H * W
    k = int(np.clip(int(L * _TOPK_FRAC), 1, C))
    idx_bits = max(int(L - 1).bit_length(), 1)

    rows = _ROWS_PER_BLOCK
    flat = Attention_map.reshape(B, L)

    out = pl.pallas_call(
        functools.partial(
            _flip_topk_kernel, k=k, rows=rows, length=L, idx_bits=idx_bits
        ),
        grid=(B // rows,),
        in_specs=[pl.BlockSpec((rows, L), lambda i: (i, 0))],
        out_specs=pl.BlockSpec((rows, L), lambda i: (i, 0)),
        out_shape=jax.ShapeDtypeStruct((B, L), jnp.float32),
        compiler_params=pltpu.CompilerParams(
            dimension_semantics=("parallel",),
        ),
    )(flat)
    return out.reshape(B, C, H, W)


# speedup vs baseline: 39.4322x; 1.0821x over previous
import functools

import jax
import jax.numpy as jnp
import numpy as np
from jax.experimental import pallas as pl
from jax.experimental.pallas import tpu as pltpu

_TOPK_FRAC = 0.2
_ROWS_PER_BLOCK = 64


def _flip_topk_kernel(x_ref, o_ref, pfx_ref, *, k, rows, length, idx_bits):
    x = x_ref[...]
    b = jax.lax.bitcast_convert_type(x, jnp.int32)
    s = jnp.where(b < 0, b ^ jnp.int32(0x7FFFFFFF), b)

    kk = jnp.int32(k)

    cnt_nonneg = jnp.sum((s >= 0).astype(jnp.int32), axis=1, keepdims=True)
    base = jnp.where(cnt_nonneg >= kk, jnp.int32(0), jnp.int32(-(2**31)))

    def value_step(i, prefix):
        bit = jnp.int32(1) << (30 - i)
        cand = base | prefix | bit
        cnt = jnp.sum((s >= cand).astype(jnp.int32), axis=1, keepdims=True)
        return jnp.where(cnt >= kk, prefix | bit, prefix)

    prefix = jax.lax.fori_loop(
        0, 31, value_step, jnp.zeros((rows, 1), jnp.int32)
    )
    t = base | prefix

    gt = s > t
    eq = s == t
    cnt_gt = jnp.sum(gt.astype(jnp.int32), axis=1, keepdims=True)
    cnt_eq = jnp.sum(eq.astype(jnp.int32), axis=1, keepdims=True)
    need = kk - cnt_gt

    ridx = jnp.int32(length - 1) - jax.lax.broadcasted_iota(
        jnp.int32, (rows, length), 1
    )
    keys = jnp.where(eq, ridx, jnp.int32(-1))

    pfx_ref[...] = jnp.zeros((rows, 1), jnp.int32)

    @pl.when(jnp.any(cnt_eq != need))
    def _tie_descent():
        def idx_step(i, pfx):
            bit = jnp.int32(1) << (idx_bits - 1 - i)
            cand = pfx | bit
            cnt = jnp.sum((keys >= cand).astype(jnp.int32), axis=1,
                          keepdims=True)
            return jnp.where(cnt >= need, pfx | bit, pfx)

        pfx_ref[...] = jax.lax.fori_loop(
            0, idx_bits, idx_step, jnp.zeros((rows, 1), jnp.int32)
        )

    flip = gt | (eq & (keys >= pfx_ref[...]))
    o_ref[...] = jnp.where(flip, 1.0 - x, x)


def kernel(Attention_map):
    B, C, H, W = Attention_map.shape
    L = C * H * W
    k = int(np.clip(int(L * _TOPK_FRAC), 1, C))
    idx_bits = max(int(L - 1).bit_length(), 1)

    rows = _ROWS_PER_BLOCK
    flat = Attention_map.reshape(B, L)

    out = pl.pallas_call(
        functools.partial(
            _flip_topk_kernel, k=k, rows=rows, length=L, idx_bits=idx_bits
        ),
        grid=(B // rows,),
        in_specs=[pl.BlockSpec((rows, L), lambda i: (i, 0))],
        out_specs=pl.BlockSpec((rows, L), lambda i: (i, 0)),
        out_shape=jax.ShapeDtypeStruct((B, L), jnp.float32),
        scratch_shapes=[pltpu.VMEM((rows, 1), jnp.int32)],
        compiler_params=pltpu.CompilerParams(
            dimension_semantics=("parallel",),
        ),
    )(flat)
    return out.reshape(B, C, H, W)
